# Initial kernel scaffold; baseline (speedup 1.0000x reference)
import jax
import jax.numpy as jnp
from jax.experimental import pallas as pl


def kernel(pos_seq, table):
    raise NotImplementedError("write your pallas kernel here")


# speedup vs baseline: 5.7836x; 5.7836x over previous
import functools

import jax
import jax.numpy as jnp
from jax import lax
from jax.experimental import pallas as pl
from jax.experimental.pallas import tpu as pltpu
from jax.experimental.pallas import tpu_sc as plsc

_GATHER_W = 128
_TASK = 256
_BLK = 1024


@functools.lru_cache(maxsize=None)
def _build(n_flat: int, n_rows: int, dim: int):
    info = plsc.get_sparse_core_info()
    nc, ns = info.num_cores, info.num_subcores
    nw = nc * ns
    per_w = n_flat // nw
    n_blk = per_w // _BLK
    tasks_per_blk = _BLK // _TASK
    ng = _TASK // _GATHER_W
    blk_rows = _BLK // _GATHER_W
    mesh = plsc.VectorSubcoreMesh(core_axis_name="c", subcore_axis_name="s")

    @functools.partial(
        pl.kernel,
        mesh=mesh,
        out_type=jax.ShapeDtypeStruct((n_flat, dim), jnp.float32),
        compiler_params=pltpu.CompilerParams(use_tc_tiling_on_sc=False),
        scratch_types=[
            pltpu.VMEM_SHARED((n_rows, dim), jnp.float32),
            pltpu.VMEM((2, blk_rows, _GATHER_W), jnp.int32),
            pltpu.VMEM((2, _TASK, dim), jnp.float32),
            pltpu.SemaphoreType.DMA,
            pltpu.SemaphoreType.DMA,
            pltpu.SemaphoreType.DMA,
            pltpu.SemaphoreType.DMA,
            pltpu.SemaphoreType.DMA,
        ],
    )
    def k(idx_hbm, table_hbm, out_hbm, table_v, idx_v, rows_v, isem,
          gsem0, gsem1, ssem0, ssem1):
        gsems = (gsem0, gsem1)
        ssems = (ssem0, ssem1)
        wid = lax.axis_index("s") * nc + lax.axis_index("c")
        base = wid * per_w

        @pl.when(lax.axis_index("s") == 0)
        def _():
            pltpu.sync_copy(table_hbm, table_v)
        plsc.subcore_barrier()
        pltpu.sync_copy(
            idx_hbm.at[pl.ds(pl.multiple_of(base // _GATHER_W, blk_rows),
                             blk_rows)],
            idx_v.at[0],
        )

        def fire_gathers(slot, blk_slot, h):
            return [
                pltpu.async_copy(
                    table_v.at[idx_v.at[blk_slot, h * ng + j]],
                    rows_v.at[slot, pl.ds(j * _GATHER_W, _GATHER_W)],
                    gsems[slot],
                )
                for j in range(ng)
            ]

        def store_copy(slot, g, h):
            off = pl.multiple_of(base + g * _BLK + h * _TASK, _TASK)
            return pltpu.make_async_copy(
                rows_v.at[slot], out_hbm.at[pl.ds(off, _TASK)], ssems[slot])

        def half_body(g, blk_slot):
            @pl.when(g + 1 < n_blk)
            def _():
                off = pl.multiple_of((base + (g + 1) * _BLK) // _GATHER_W,
                                     blk_rows)
                pltpu.async_copy(idx_hbm.at[pl.ds(off, blk_rows)],
                                 idx_v.at[1 - blk_slot], isem)

            for h in range(tasks_per_blk):
                slot = h % 2
                @pl.when(jnp.logical_or(g > 0, h >= 2))
                def _(slot=slot, h=h):
                    prev_g = g - 1 if h < 2 else g
                    prev_h = h + tasks_per_blk - 2 if h < 2 else h - 2
                    store_copy(slot, prev_g, prev_h).wait()
                copies = fire_gathers(slot, blk_slot, h)
                for c in copies:
                    c.wait()
                store_copy(slot, g, h).start()

            @pl.when(g + 1 < n_blk)
            def _():
                pltpu.make_async_copy(
                    idx_hbm.at[pl.ds(0, blk_rows)], idx_v.at[1 - blk_slot],
                    isem).wait()

        def body(gg, carry):
            half_body(gg * 2, 0)
            half_body(gg * 2 + 1, 1)
            return carry

        lax.fori_loop(0, n_blk // 2, body, 0)
        store_copy(0, n_blk - 1, tasks_per_blk - 2).wait()
        store_copy(1, n_blk - 1, tasks_per_blk - 1).wait()

    return k


def kernel(pos_seq, table):
    b, s = pos_seq.shape
    n_rows, dim = table.shape
    n_flat = b * s
    table_eff = table.at[0].set(0.0)
    idx2d = pos_seq.astype(jnp.int32).reshape(n_flat // _GATHER_W, _GATHER_W)
    out = _build(n_flat, n_rows, dim)(idx2d, table_eff)
    return out.reshape(b, s, dim)


# speedup vs baseline: 5.8235x; 1.0069x over previous
import functools

import jax
import jax.numpy as jnp
from jax import lax
from jax.experimental import pallas as pl
from jax.experimental.pallas import tpu as pltpu
from jax.experimental.pallas import tpu_sc as plsc

_TASK_B = 4
_BLK_B = 16


@functools.lru_cache(maxsize=None)
def _build(b: int, s: int, n_rows: int, dim: int):
    info = plsc.get_sparse_core_info()
    nc, ns = info.num_cores, info.num_subcores
    nw = nc * ns
    per_w = b // nw
    n_blk = per_w // _BLK_B
    tasks_per_blk = _BLK_B // _TASK_B
    splits = [(o, min(128, s - o)) for o in range(0, s, 128)]
    mesh = plsc.VectorSubcoreMesh(core_axis_name="c", subcore_axis_name="s")

    @functools.partial(
        pl.kernel,
        mesh=mesh,
        out_type=jax.ShapeDtypeStruct((b, s, dim), jnp.float32),
        compiler_params=pltpu.CompilerParams(use_tc_tiling_on_sc=False),
        scratch_types=[
            pltpu.VMEM_SHARED((n_rows, dim), jnp.float32),
            pltpu.VMEM((2, _BLK_B, s), jnp.int32),
            pltpu.VMEM((2, _TASK_B, s, dim), jnp.float32),
            pltpu.SemaphoreType.DMA,
            pltpu.SemaphoreType.DMA,
            pltpu.SemaphoreType.DMA,
            pltpu.SemaphoreType.DMA,
            pltpu.SemaphoreType.DMA,
        ],
    )
    def k(idx_hbm, table_hbm, out_hbm, table_v, idx_v, rows_v, isem,
          gsem0, gsem1, ssem0, ssem1):
        gsems = (gsem0, gsem1)
        ssems = (ssem0, ssem1)
        wid = lax.axis_index("s") * nc + lax.axis_index("c")
        base = wid * per_w

        @pl.when(lax.axis_index("s") == 0)
        def _():
            pltpu.sync_copy(table_hbm, table_v)
        plsc.subcore_barrier()
        pltpu.sync_copy(idx_hbm.at[pl.ds(pl.multiple_of(base, _BLK_B),
                                         _BLK_B)], idx_v.at[0])

        def fire_gathers(slot, blk_slot, h):
            copies = []
            for r in range(_TASK_B):
                for (o, w) in splits:
                    copies.append(pltpu.async_copy(
                        table_v.at[idx_v.at[blk_slot, h * _TASK_B + r,
                                            pl.ds(o, w)]],
                        rows_v.at[slot, r, pl.ds(o, w)],
                        gsems[slot],
                    ))
            return copies

        def store_copy(slot, g, h):
            row0 = pl.multiple_of(base + g * _BLK_B + h * _TASK_B, _TASK_B)
            return pltpu.make_async_copy(
                rows_v.at[slot], out_hbm.at[pl.ds(row0, _TASK_B)],
                ssems[slot])

        def half_body(g, blk_slot):
            @pl.when(g + 1 < n_blk)
            def _():
                row0 = pl.multiple_of(base + (g + 1) * _BLK_B, _BLK_B)
                pltpu.async_copy(idx_hbm.at[pl.ds(row0, _BLK_B)],
                                 idx_v.at[1 - blk_slot], isem)

            for h in range(tasks_per_blk):
                slot = h % 2
                @pl.when(jnp.logical_or(g > 0, h >= 2))
                def _(slot=slot, h=h):
                    prev_g = g - 1 if h < 2 else g
                    prev_h = h + tasks_per_blk - 2 if h < 2 else h - 2
                    store_copy(slot, prev_g, prev_h).wait()
                copies = fire_gathers(slot, blk_slot, h)
                for c in copies:
                    c.wait()
                store_copy(slot, g, h).start()

            @pl.when(g + 1 < n_blk)
            def _():
                pltpu.make_async_copy(
                    idx_hbm.at[pl.ds(0, _BLK_B)], idx_v.at[1 - blk_slot],
                    isem).wait()

        def body(gg, carry):
            half_body(gg * 2, 0)
            half_body(gg * 2 + 1, 1)
            return carry

        lax.fori_loop(0, n_blk // 2, body, 0)
        store_copy(0, n_blk - 1, tasks_per_blk - 2).wait()
        store_copy(1, n_blk - 1, tasks_per_blk - 1).wait()

    return k


def kernel(pos_seq, table):
    b, s = pos_seq.shape
    n_rows, dim = table.shape
    table_eff = table.at[0].set(0.0)
    return _build(b, s, n_rows, dim)(pos_seq.astype(jnp.int32), table_eff)


# speedup vs baseline: 9.8566x; 1.6926x over previous
import functools

import jax
import jax.numpy as jnp
from jax import lax
from jax.experimental import pallas as pl
from jax.experimental.pallas import tpu as pltpu
from jax.experimental.pallas import tpu_sc as plsc

_TASK_B = 2
_BLK_B = 16
_PAD_D = 128


@functools.lru_cache(maxsize=None)
def _build(b: int, s: int, n_rows: int):
    info = plsc.get_sparse_core_info()
    nc, ns = info.num_cores, info.num_subcores
    nw = nc * ns
    per_w = b // nw
    n_blk = per_w // _BLK_B
    tasks_per_blk = _BLK_B // _TASK_B
    splits = [(o, min(128, s - o)) for o in range(0, s, 128)]
    mesh = plsc.VectorSubcoreMesh(core_axis_name="c", subcore_axis_name="s")

    @functools.partial(
        pl.kernel,
        mesh=mesh,
        out_type=jax.ShapeDtypeStruct((b, s, _PAD_D), jnp.float32),
        scratch_types=[
            pltpu.VMEM_SHARED((n_rows, _PAD_D), jnp.float32),
            pltpu.VMEM((2, _BLK_B, s), jnp.int32),
            pltpu.VMEM((2, _TASK_B, s, _PAD_D), jnp.float32),
            pltpu.SemaphoreType.DMA,
            pltpu.SemaphoreType.DMA,
            pltpu.SemaphoreType.DMA,
            pltpu.SemaphoreType.DMA,
            pltpu.SemaphoreType.DMA,
        ],
    )
    def k(idx_hbm, table_hbm, out_hbm, table_v, idx_v, rows_v, isem,
          gsem0, gsem1, ssem0, ssem1):
        gsems = (gsem0, gsem1)
        ssems = (ssem0, ssem1)
        wid = lax.axis_index("s") * nc + lax.axis_index("c")
        base = wid * per_w

        @pl.when(lax.axis_index("s") == 0)
        def _():
            pltpu.sync_copy(table_hbm, table_v)
        plsc.subcore_barrier()
        pltpu.sync_copy(idx_hbm.at[pl.ds(pl.multiple_of(base, _BLK_B),
                                         _BLK_B)], idx_v.at[0])

        def fire_gathers(slot, blk_slot, h):
            copies = []
            for r in range(_TASK_B):
                for (o, w) in splits:
                    copies.append(pltpu.async_copy(
                        table_v.at[idx_v.at[blk_slot, h * _TASK_B + r,
                                            pl.ds(o, w)]],
                        rows_v.at[slot, r, pl.ds(o, w)],
                        gsems[slot],
                    ))
            return copies

        def store_copy(slot, g, h):
            row0 = pl.multiple_of(base + g * _BLK_B + h * _TASK_B, _TASK_B)
            return pltpu.make_async_copy(
                rows_v.at[slot], out_hbm.at[pl.ds(row0, _TASK_B)],
                ssems[slot])

        def half_body(g, blk_slot):
            @pl.when(g + 1 < n_blk)
            def _():
                row0 = pl.multiple_of(base + (g + 1) * _BLK_B, _BLK_B)
                pltpu.async_copy(idx_hbm.at[pl.ds(row0, _BLK_B)],
                                 idx_v.at[1 - blk_slot], isem)

            for h in range(tasks_per_blk):
                slot = h % 2
                @pl.when(jnp.logical_or(g > 0, h >= 2))
                def _(slot=slot, h=h):
                    prev_g = g - 1 if h < 2 else g
                    prev_h = h + tasks_per_blk - 2 if h < 2 else h - 2
                    store_copy(slot, prev_g, prev_h).wait()
                copies = fire_gathers(slot, blk_slot, h)
                for c in copies:
                    c.wait()
                store_copy(slot, g, h).start()

            @pl.when(g + 1 < n_blk)
            def _():
                pltpu.make_async_copy(
                    idx_hbm.at[pl.ds(0, _BLK_B)], idx_v.at[1 - blk_slot],
                    isem).wait()

        def body(gg, carry):
            half_body(gg * 2, 0)
            half_body(gg * 2 + 1, 1)
            return carry

        lax.fori_loop(0, n_blk // 2, body, 0)
        store_copy(0, n_blk - 1, tasks_per_blk - 2).wait()
        store_copy(1, n_blk - 1, tasks_per_blk - 1).wait()

    return k


def kernel(pos_seq, table):
    b, s = pos_seq.shape
    n_rows, dim = table.shape
    table_eff = jnp.pad(table.at[0].set(0.0), ((0, 0), (0, _PAD_D - dim)))
    out = _build(b, s, n_rows)(pos_seq.astype(jnp.int32), table_eff)
    return out[:, :, :dim]
